# baseline (speedup 1.0000x reference)
import functools

import jax
import jax.numpy as jnp
from jax import lax
from jax.experimental import pallas as pl
from jax.experimental.pallas import tpu as pltpu
from jax.experimental.pallas import tpu_sc as plsc


_CH = 128
_K = 2
_NBUF = 5
_LAG = 3


@functools.lru_cache(maxsize=None)
def _build(n_workers: int, n_cores: int, b_tot: int, vocab: int, d: int):
    b_per_w = b_tot // n_workers
    n_ch = b_per_w // _CH
    n_grp = n_ch // _K
    rounds = n_grp // _NBUF
    grp = _K * _CH
    mesh = plsc.VectorSubcoreMesh(core_axis_name="c", subcore_axis_name="s")

    scratch = [pltpu.VMEM((n_ch, _CH), jnp.int32)]
    scratch += [pltpu.VMEM((grp, d), jnp.float32) for _ in range(_NBUF)]
    scratch += [pltpu.SemaphoreType.DMA for _ in range(2 * _NBUF)]

    @functools.partial(
        pl.kernel,
        mesh=mesh,
        out_type=jax.ShapeDtypeStruct((b_tot, d), jnp.float32),
        compiler_params=pltpu.CompilerParams(use_tc_tiling_on_sc=False),
        scratch_types=scratch,
    )
    def emb(x_hbm, table_hbm, out_hbm, idx_v, *rest):
        bufs = rest[:_NBUF]
        gsem = rest[_NBUF:2 * _NBUF]
        wsem = rest[2 * _NBUF:]
        wid = lax.axis_index("s") * n_cores + lax.axis_index("c")
        base = wid * b_per_w
        pltpu.sync_copy(x_hbm.at[wid], idx_v)

        def g_copies(g, b):
            return [
                pltpu.make_async_copy(
                    table_hbm.at[idx_v.at[g * _K + j]],
                    bufs[b].at[pl.ds(j * _CH, _CH)], gsem[b])
                for j in range(_K)
            ]

        def w_copy(g, b):
            return pltpu.make_async_copy(
                bufs[b], out_hbm.at[pl.ds(base + g * grp, grp)], wsem[b])

        for b in range(_LAG):
            for cp in g_copies(b, b):
                cp.start()

        def body(r, carry):
            g0 = r * _NBUF
            for i in range(_NBUF):
                g = g0 + i
                fg = g + _LAG
                fb = (i + _LAG) % _NBUF

                @pl.when(fg < n_grp)
                def _():
                    @pl.when(fg >= _NBUF)
                    def _():
                        w_copy(fg - _NBUF, fb).wait()
                    for cp in g_copies(fg, fb):
                        cp.start()

                for cp in g_copies(g, i):
                    cp.wait()
                w_copy(g, i).start()
            return carry

        lax.fori_loop(0, rounds, body, 0)

        for b in range(_NBUF):
            w_copy(n_grp - _NBUF + b, (n_grp - _NBUF + b) % _NBUF).wait()

    return emb


def kernel(x, table):
    b, s = x.shape
    vocab, d = table.shape
    info = plsc.get_sparse_core_info()
    n_workers = info.num_cores * info.num_subcores
    b_tot = b * s
    b_per_w = b_tot // n_workers
    xf = x.reshape(n_workers, b_per_w // _CH, _CH)
    emb = _build(n_workers, info.num_cores, b_tot, vocab, d)
    out = emb(xf, table)
    return out.reshape(b, s, d)
